# baseline (speedup 1.0000x reference)
import functools

import jax
import jax.numpy as jnp
from jax import lax
from jax.experimental import pallas as pl
from jax.experimental.pallas import tpu as pltpu
from jax.experimental.pallas import tpu_sc as plsc

NC = 2
NS = 16
NW = NC * NS
CH = 64
IB = 40
KB = 4
LA = 2


def _sc_mesh():
    return plsc.VectorSubcoreMesh(core_axis_name="c", subcore_axis_name="s")


def _make_deg_kernel(n_pad, nch):
    trows = n_pad // NS

    @functools.partial(
        pl.kernel,
        out_type=jax.ShapeDtypeStruct((NC, n_pad), jnp.float32),
        mesh=_sc_mesh(),
        scratch_types=[
            pltpu.VMEM((nch, CH), jnp.int32),
            pltpu.VMEM((CH,), jnp.float32),
            pltpu.VMEM_SHARED((n_pad,), jnp.float32),
            pltpu.SemaphoreType.DMA,
        ],
    )
    def deg_kernel(dstw_hbm, zeros_hbm, out_hbm, idx_v, ones_v, acc, dsem):
        c = lax.axis_index("c")
        s = lax.axis_index("s")
        wid = c * NS + s
        pltpu.sync_copy(dstw_hbm.at[wid], idx_v)
        for i in range(CH // 16):
            ones_v[pl.ds(i * 16, 16)] = jnp.ones((16,), jnp.float32)
        pltpu.sync_copy(zeros_hbm.at[pl.ds(s * trows, trows)],
                        acc.at[pl.ds(s * trows, trows)])
        plsc.subcore_barrier()

        gr = 8

        def body(g, carry):
            for b in range(gr):
                pltpu.async_copy(ones_v, acc.at[idx_v.at[g * gr + b]], dsem,
                                 add=True)
            for b in range(gr):
                pltpu.make_async_copy(ones_v, acc.at[idx_v.at[g * gr + b]],
                                      dsem).wait()
            return carry

        lax.fori_loop(0, nch // gr, body, 0)
        plsc.subcore_barrier()
        pltpu.sync_copy(acc.at[pl.ds(s * trows, trows)],
                        out_hbm.at[c].at[pl.ds(s * trows, trows)])

    return deg_kernel


def _make_agg_kernel(n_pad, nch, d):
    trows = n_pad // NS
    nblk = nch // IB

    @functools.partial(
        pl.kernel,
        out_type=jax.ShapeDtypeStruct((NC, n_pad, d), jnp.float32),
        mesh=_sc_mesh(),
        scratch_types=[
            pltpu.VMEM((IB, CH), jnp.int32),
            pltpu.VMEM((IB, CH), jnp.int32),
            pltpu.VMEM((CH, d), jnp.float32),
            pltpu.VMEM((CH, d), jnp.float32),
            pltpu.VMEM((CH, d), jnp.float32),
            pltpu.VMEM((CH, d), jnp.float32),
            pltpu.VMEM_SHARED((n_pad, d), jnp.float32),
            pltpu.SemaphoreType.DMA,
            pltpu.SemaphoreType.DMA,
            pltpu.SemaphoreType.DMA,
            pltpu.SemaphoreType.DMA,
            pltpu.SemaphoreType.DMA,
            pltpu.SemaphoreType.DMA,
            pltpu.SemaphoreType.DMA,
            pltpu.SemaphoreType.DMA,
            pltpu.SemaphoreType.DMA,
            pltpu.SemaphoreType.DMA,
        ],
    )
    def agg_kernel(hp_hbm, srcw_hbm, dstw_hbm, zeros_hbm, out_hbm,
                   src_i, dst_i, b0, b1, b2, b3, acc,
                   g0, g1, g2, g3, s0, s1, s2, s3, zsem, isem):
        bufs = (b0, b1, b2, b3)
        gsems = (g0, g1, g2, g3)
        ssems = (s0, s1, s2, s3)
        c = lax.axis_index("c")
        s = lax.axis_index("s")
        wid = c * NS + s
        def load_blk(blk):
            pltpu.async_copy(srcw_hbm.at[wid].at[pl.ds(blk * IB, IB)],
                             src_i, isem)
            pltpu.async_copy(dstw_hbm.at[wid].at[pl.ds(blk * IB, IB)],
                             dst_i, isem)
            pltpu.make_async_copy(srcw_hbm.at[wid].at[pl.ds(blk * IB, IB)],
                                  src_i, isem).wait()
            pltpu.make_async_copy(dstw_hbm.at[wid].at[pl.ds(blk * IB, IB)],
                                  dst_i, isem).wait()

        pltpu.async_copy(zeros_hbm, acc.at[pl.ds(s * trows, trows)], zsem)
        load_blk(0)
        pltpu.async_copy(hp_hbm.at[src_i.at[0]], bufs[0], gsems[0])
        pltpu.async_copy(hp_hbm.at[src_i.at[1]], bufs[1], gsems[1])
        pltpu.make_async_copy(zeros_hbm, acc.at[pl.ds(s * trows, trows)],
                              zsem).wait()
        plsc.subcore_barrier()

        for blk in range(nblk):
            src_v = src_i
            dst_v = dst_i
            if blk > 0:
                load_blk(blk)
                for b in range(LA):
                    pltpu.async_copy(hp_hbm.at[src_v.at[b]], bufs[b],
                                     gsems[b])

            def body(g, carry):
                for b in range(KB):
                    j = g * KB + b
                    jj = j + LA
                    bb = (b + LA) % KB

                    @pl.when(jj < IB)
                    def _():
                        @pl.when(j >= LA)
                        def _():
                            pltpu.make_async_copy(
                                bufs[bb], acc.at[dst_v.at[j - LA]],
                                ssems[bb]).wait()

                        pltpu.async_copy(hp_hbm.at[src_v.at[jj]], bufs[bb],
                                         gsems[bb])

                    pltpu.make_async_copy(hp_hbm.at[src_v.at[j]], bufs[b],
                                          gsems[b]).wait()
                    pltpu.async_copy(bufs[b], acc.at[dst_v.at[j]], ssems[b],
                                     add=True)
                return carry

            lax.fori_loop(0, IB // KB, body, 0)
            for j in range(IB - KB, IB):
                b = j % KB
                pltpu.make_async_copy(bufs[b], acc.at[dst_v.at[j]],
                                      ssems[b]).wait()
        plsc.subcore_barrier()
        pltpu.sync_copy(acc.at[pl.ds(s * trows, trows)],
                        out_hbm.at[c].at[pl.ds(s * trows, trows)])

    return agg_kernel


def _matmul_body(x_ref, w_ref, out_ref):
    out_ref[...] = jnp.dot(x_ref[...], w_ref[...],
                           preferred_element_type=jnp.float32,
                           precision=lax.Precision.HIGHEST)


def _scale(t, deg_ref):
    n_pad, d = t.shape
    g = n_pad // 128
    dis = lax.rsqrt(1.0 + deg_ref[0] + deg_ref[1])
    return (t.reshape(g, 128, d) * dis[:, :, None]).reshape(n_pad, d)


def _hprime_body(h_ref, deg_ref, out_ref):
    out_ref[...] = _scale(h_ref[...], deg_ref)


def _base_body(hp_ref, deg_ref, bgc_ref, wfc_ref, bfc_ref, out_ref):
    t = _scale(hp_ref[...], deg_ref) + bgc_ref[...]
    out_ref[...] = jnp.dot(t, wfc_ref[...],
                           preferred_element_type=jnp.float32,
                           precision=lax.Precision.HIGHEST) + bfc_ref[...]


def _out_body(s_ref, deg_ref, wfc_ref, base_ref, out_ref):
    t = _scale(s_ref[0] + s_ref[1], deg_ref)
    out_ref[...] = jnp.dot(t, wfc_ref[...],
                           preferred_element_type=jnp.float32,
                           precision=lax.Precision.HIGHEST) + base_ref[...]


def kernel(x, edge_index_adj, W_gc, b_gc, W_fc, b_fc):
    n, d_in = x.shape
    d_h = W_gc.shape[1]
    d_out = W_fc.shape[1]
    e = edge_index_adj.shape[1]

    n_pad = -(-n // 2048) * 2048
    epw = -(-e // (NW * CH * IB)) * CH * IB
    e_pad = epw * NW
    nch = epw // CH

    src = edge_index_adj[0]
    dst = edge_index_adj[1]
    pad = e_pad - e
    if pad:
        ar = jnp.arange(pad, dtype=jnp.int32)
        src = jnp.concatenate([src, n + ar % (n_pad - n)])
        dst_deg = jnp.concatenate([dst, n + ar % (n_pad - n)])
        dst = jnp.concatenate([dst, (ar * 997) % n_pad])
    else:
        dst_deg = dst
    srcw = src.reshape(NW, nch, CH)
    dstw = dst.reshape(NW, nch, CH)
    dstw_deg = dst_deg.reshape(NW, nch, CH)

    x_pad = jnp.pad(x, ((0, n_pad - n), (0, 0)))
    zeros1 = jnp.zeros((n_pad,), jnp.float32)
    zeros2 = jnp.zeros((n_pad // NS, d_h), jnp.float32)

    h = pl.pallas_call(
        _matmul_body,
        out_shape=jax.ShapeDtypeStruct((n_pad, d_h), jnp.float32),
    )(x_pad, W_gc)

    deg_parts = _make_deg_kernel(n_pad, nch)(dstw_deg, zeros1)
    deg3 = deg_parts.reshape(NC, n_pad // 128, 128)

    hp = pl.pallas_call(
        _hprime_body,
        out_shape=jax.ShapeDtypeStruct((n_pad, d_h), jnp.float32),
    )(h, deg3)

    s_parts = _make_agg_kernel(n_pad, nch, d_h)(hp, srcw, dstw, zeros2)

    base = pl.pallas_call(
        _base_body,
        out_shape=jax.ShapeDtypeStruct((n_pad, d_out), jnp.float32),
    )(hp, deg3, b_gc, W_fc, b_fc)

    out = pl.pallas_call(
        _out_body,
        out_shape=jax.ShapeDtypeStruct((n_pad, d_out), jnp.float32),
    )(s_parts, deg3, W_fc, base)

    return out[:n]


# speedup vs baseline: 1.0387x; 1.0387x over previous
import functools

import jax
import jax.numpy as jnp
from jax import lax
from jax.experimental import pallas as pl
from jax.experimental.pallas import tpu as pltpu
from jax.experimental.pallas import tpu_sc as plsc

NC = 2
NS = 16
NW = NC * NS
CH = 64
IB = 40
KB = 4
LA = 2


def _sc_mesh():
    return plsc.VectorSubcoreMesh(core_axis_name="c", subcore_axis_name="s")


def _make_deg_kernel(n_pad, nch):
    trows = n_pad // NS

    @functools.partial(
        pl.kernel,
        out_type=jax.ShapeDtypeStruct((NC, n_pad), jnp.float32),
        mesh=_sc_mesh(),
        scratch_types=[
            pltpu.VMEM((nch, CH), jnp.int32),
            pltpu.VMEM((CH,), jnp.float32),
            pltpu.VMEM_SHARED((n_pad,), jnp.float32),
            pltpu.SemaphoreType.DMA,
        ],
    )
    def deg_kernel(dstw_hbm, zeros_hbm, out_hbm, idx_v, ones_v, acc, dsem):
        c = lax.axis_index("c")
        s = lax.axis_index("s")
        wid = c * NS + s
        pltpu.sync_copy(dstw_hbm.at[wid], idx_v)
        for i in range(CH // 16):
            ones_v[pl.ds(i * 16, 16)] = jnp.ones((16,), jnp.float32)
        pltpu.sync_copy(zeros_hbm.at[pl.ds(s * trows, trows)],
                        acc.at[pl.ds(s * trows, trows)])
        plsc.subcore_barrier()

        gr = 8

        def body(g, carry):
            for b in range(gr):
                pltpu.async_copy(ones_v, acc.at[idx_v.at[g * gr + b]], dsem,
                                 add=True)
            for b in range(gr):
                pltpu.make_async_copy(ones_v, acc.at[idx_v.at[g * gr + b]],
                                      dsem).wait()
            return carry

        lax.fori_loop(0, nch // gr, body, 0)
        plsc.subcore_barrier()
        pltpu.sync_copy(acc.at[pl.ds(s * trows, trows)],
                        out_hbm.at[c].at[pl.ds(s * trows, trows)])

    return deg_kernel


def _make_agg_kernel(n_pad, nch, d):
    trows = n_pad // NS
    nblk = nch // IB

    @functools.partial(
        pl.kernel,
        out_type=jax.ShapeDtypeStruct((NC, n_pad, d), jnp.float32),
        mesh=_sc_mesh(),
        scratch_types=[
            pltpu.VMEM((IB, CH), jnp.int32),
            pltpu.VMEM((IB, CH), jnp.int32),
            pltpu.VMEM((CH, d), jnp.float32),
            pltpu.VMEM((CH, d), jnp.float32),
            pltpu.VMEM((CH, d), jnp.float32),
            pltpu.VMEM((CH, d), jnp.float32),
            pltpu.VMEM_SHARED((n_pad, d), jnp.float32),
            pltpu.SemaphoreType.DMA,
            pltpu.SemaphoreType.DMA,
            pltpu.SemaphoreType.DMA,
            pltpu.SemaphoreType.DMA,
            pltpu.SemaphoreType.DMA,
            pltpu.SemaphoreType.DMA,
            pltpu.SemaphoreType.DMA,
            pltpu.SemaphoreType.DMA,
            pltpu.SemaphoreType.DMA,
            pltpu.SemaphoreType.DMA,
        ],
    )
    def agg_kernel(hp_hbm, srcw_hbm, dstw_hbm, zeros_hbm, out_hbm,
                   src_i, dst_i, b0, b1, b2, b3, acc,
                   g0, g1, g2, g3, s0, s1, s2, s3, zsem, isem):
        bufs = (b0, b1, b2, b3)
        gsems = (g0, g1, g2, g3)
        ssems = (s0, s1, s2, s3)
        c = lax.axis_index("c")
        s = lax.axis_index("s")
        wid = c * NS + s
        def load_blk(blk):
            pltpu.async_copy(srcw_hbm.at[wid].at[pl.ds(blk * IB, IB)],
                             src_i, isem)
            pltpu.async_copy(dstw_hbm.at[wid].at[pl.ds(blk * IB, IB)],
                             dst_i, isem)
            pltpu.make_async_copy(srcw_hbm.at[wid].at[pl.ds(blk * IB, IB)],
                                  src_i, isem).wait()
            pltpu.make_async_copy(dstw_hbm.at[wid].at[pl.ds(blk * IB, IB)],
                                  dst_i, isem).wait()

        pltpu.async_copy(zeros_hbm, acc.at[pl.ds(s * trows, trows)], zsem)
        load_blk(0)
        pltpu.async_copy(hp_hbm.at[src_i.at[0]], bufs[0], gsems[0])
        pltpu.async_copy(hp_hbm.at[src_i.at[1]], bufs[1], gsems[1])
        pltpu.make_async_copy(zeros_hbm, acc.at[pl.ds(s * trows, trows)],
                              zsem).wait()
        plsc.subcore_barrier()

        for blk in range(nblk):
            src_v = src_i
            dst_v = dst_i
            if blk > 0:
                load_blk(blk)
                for b in range(LA):
                    pltpu.async_copy(hp_hbm.at[src_v.at[b]], bufs[b],
                                     gsems[b])

            def body(g, carry):
                for b in range(KB):
                    j = g * KB + b
                    jj = j + LA
                    bb = (b + LA) % KB

                    @pl.when(jj < IB)
                    def _():
                        @pl.when(j >= LA)
                        def _():
                            pltpu.make_async_copy(
                                bufs[bb], acc.at[dst_v.at[j - LA]],
                                ssems[bb]).wait()

                        pltpu.async_copy(hp_hbm.at[src_v.at[jj]], bufs[bb],
                                         gsems[bb])

                    pltpu.make_async_copy(hp_hbm.at[src_v.at[j]], bufs[b],
                                          gsems[b]).wait()
                    pltpu.async_copy(bufs[b], acc.at[dst_v.at[j]], ssems[b],
                                     add=True)
                return carry

            lax.fori_loop(0, IB // KB, body, 0)
            for j in range(IB - KB, IB):
                b = j % KB
                pltpu.make_async_copy(bufs[b], acc.at[dst_v.at[j]],
                                      ssems[b]).wait()
        plsc.subcore_barrier()
        pltpu.sync_copy(acc.at[pl.ds(s * trows, trows)],
                        out_hbm.at[c].at[pl.ds(s * trows, trows)])

    return agg_kernel


def _scale(t, deg_ref):
    n_pad, d = t.shape
    g = n_pad // 128
    dis = lax.rsqrt(1.0 + deg_ref[0] + deg_ref[1])
    return (t.reshape(g, 128, d) * dis[:, :, None]).reshape(n_pad, d)


def _hprime_body(x_ref, w_ref, deg_ref, out_ref):
    h = jnp.dot(x_ref[...], w_ref[...],
                preferred_element_type=jnp.float32,
                precision=lax.Precision.HIGHEST)
    out_ref[...] = _scale(h, deg_ref)


def _out_body(s_ref, hp_ref, deg_ref, bgc_ref, wfc_ref, bfc_ref, out_ref):
    t = _scale(s_ref[0] + s_ref[1] + hp_ref[...], deg_ref) + bgc_ref[...]
    out_ref[...] = jnp.dot(t, wfc_ref[...],
                           preferred_element_type=jnp.float32,
                           precision=lax.Precision.HIGHEST) + bfc_ref[...]


def kernel(x, edge_index_adj, W_gc, b_gc, W_fc, b_fc):
    n, d_in = x.shape
    d_h = W_gc.shape[1]
    d_out = W_fc.shape[1]
    e = edge_index_adj.shape[1]

    n_pad = -(-n // 2048) * 2048
    epw = -(-e // (NW * CH * IB)) * CH * IB
    e_pad = epw * NW
    nch = epw // CH

    src = edge_index_adj[0]
    dst = edge_index_adj[1]
    pad = e_pad - e
    if pad:
        ar = jnp.arange(pad, dtype=jnp.int32)
        src = jnp.concatenate([src, n + ar % (n_pad - n)])
        dst_deg = jnp.concatenate([dst, n + ar % (n_pad - n)])
        dst = jnp.concatenate([dst, (ar * 997) % n_pad])
    else:
        dst_deg = dst
    srcw = src.reshape(NW, nch, CH)
    dstw = dst.reshape(NW, nch, CH)
    dstw_deg = dst_deg.reshape(NW, nch, CH)

    x_pad = jnp.pad(x, ((0, n_pad - n), (0, 0)))
    zeros1 = jnp.zeros((n_pad,), jnp.float32)
    zeros2 = jnp.zeros((n_pad // NS, d_h), jnp.float32)

    deg_parts = _make_deg_kernel(n_pad, nch)(dstw_deg, zeros1)
    deg3 = deg_parts.reshape(NC, n_pad // 128, 128)

    hp = pl.pallas_call(
        _hprime_body,
        out_shape=jax.ShapeDtypeStruct((n_pad, d_h), jnp.float32),
    )(x_pad, W_gc, deg3)

    s_parts = _make_agg_kernel(n_pad, nch, d_h)(hp, srcw, dstw, zeros2)

    out = pl.pallas_call(
        _out_body,
        out_shape=jax.ShapeDtypeStruct((n_pad, d_out), jnp.float32),
    )(s_parts, hp, deg3, b_gc, W_fc, b_fc)

    return out[:n]
